# baseline (speedup 1.0000x reference)
import dataclasses
import functools

import jax
import jax.numpy as jnp
from jax import lax
from jax.experimental import pallas as pl
from jax.experimental.pallas import tpu as pltpu
from jax.experimental.pallas import tpu_sc as plsc

IN_F = 4096
OUT_F = 4096
RANK = 64
TOPK = 8
SCALE = 2.0

NCORES = 2
NSUB = 16
NTILES = NCORES * NSUB
LANES = 16

BT = 256
NCHUNK = 4


def _mm1_body(x_ref, at_ref, y_ref):
    y_ref[...] = jnp.dot(x_ref[...].astype(jnp.bfloat16),
                         at_ref[...].astype(jnp.bfloat16),
                         preferred_element_type=jnp.float32)


def _mm2_body(my_ref, bt_ref, o_ref):
    o_ref[...] = jnp.dot(my_ref[...].astype(jnp.bfloat16),
                         bt_ref[...].astype(jnp.bfloat16),
                         preferred_element_type=jnp.float32) * SCALE


def _make_sc_mask(n_tok):
    T = n_tok // NTILES
    G = T // LANES
    mesh = plsc.VectorSubcoreMesh(core_axis_name="c", subcore_axis_name="s")
    cp = pltpu.CompilerParams()
    if "needs_layout_passes" in pltpu.CompilerParams.__dataclass_fields__:
        cp = dataclasses.replace(cp, needs_layout_passes=False)

    @functools.partial(
        pl.kernel,
        mesh=mesh,
        compiler_params=cp,
        out_type=jax.ShapeDtypeStruct((n_tok, RANK), jnp.float32),
        scratch_types=[
            pltpu.VMEM((T, RANK), jnp.float32),
            pltpu.VMEM((RANK, T), jnp.float32),
            pltpu.VMEM((RANK, LANES), jnp.float32),
            pltpu.SemaphoreType.DMA,
        ],
    )
    def sc_mask(y_hbm, d_hbm, o_hbm, y_v, a_t, d_v, sem):
        wid = lax.axis_index("s") * NCORES + lax.axis_index("c")
        base = wid * T
        pltpu.async_copy(y_hbm.at[pl.ds(base, T)], y_v, sem).wait()
        pltpu.async_copy(d_hbm, d_v, sem).wait()
        iota = lax.broadcasted_iota(jnp.int32, (LANES,), 0)

        @pl.loop(0, G)
        def _group(g):
            row0 = g * LANES
            rows = iota + row0
            cols = [jnp.full((LANES,), r, jnp.int32) for r in range(RANK)]

            for r in range(RANK):
                v = plsc.load_gather(y_v, [rows, cols[r]])
                a_t[r, pl.ds(row0, LANES)] = jnp.abs(v + d_v[r, :])

            @pl.loop(0, TOPK)
            def _round(_):
                m = a_t[0, pl.ds(row0, LANES)]
                for r in range(1, RANK):
                    m = jnp.maximum(m, a_t[r, pl.ds(row0, LANES)])
                found = jnp.zeros((LANES,), jnp.bool_)
                for r in range(RANK):
                    v = a_t[r, pl.ds(row0, LANES)]
                    sel = (v == m) & jnp.logical_not(found)
                    found = found | sel
                    a_t[r, pl.ds(row0, LANES)] = jnp.where(sel, -1.0, v)

            for r in range(RANK):
                keep = a_t[r, pl.ds(row0, LANES)] == -1.0
                v = plsc.load_gather(y_v, [rows, cols[r]])
                plsc.store_scatter(y_v, [rows, cols[r]],
                                   jnp.where(keep, v, 0.0))

        pltpu.sync_copy(y_v, o_hbm.at[pl.ds(base, T)])

    return sc_mask


def kernel(x, A, d, B):
    orig_shape = x.shape
    xt = x.reshape(-1, IN_F)
    n_tok = xt.shape[0]
    nt = n_tok // NCHUNK

    at = A.T
    bt = B.T
    d_splat = jnp.broadcast_to(d.reshape(RANK, 1), (RANK, LANES))

    sc_mask = _make_sc_mask(nt)

    masked = []
    for c in range(NCHUNK):
        y_c = pl.pallas_call(
            _mm1_body,
            grid=(nt // BT,),
            in_specs=[
                pl.BlockSpec((BT, IN_F), lambda i, c=c: (c * (nt // BT) + i, 0)),
                pl.BlockSpec((IN_F, RANK), lambda i: (0, 0)),
            ],
            out_specs=pl.BlockSpec((BT, RANK), lambda i: (i, 0)),
            out_shape=jax.ShapeDtypeStruct((nt, RANK), jnp.float32),
            compiler_params=pltpu.CompilerParams(
                dimension_semantics=("parallel",)),
        )(xt, at)
        masked.append(sc_mask(y_c, d_splat))

    my_full = jnp.concatenate(masked, axis=0)

    out = pl.pallas_call(
        _mm2_body,
        grid=(n_tok // BT,),
        in_specs=[
            pl.BlockSpec((BT, RANK), lambda i: (i, 0)),
            pl.BlockSpec((RANK, OUT_F), lambda i: (0, 0)),
        ],
        out_specs=pl.BlockSpec((BT, OUT_F), lambda i: (i, 0)),
        out_shape=jax.ShapeDtypeStruct((n_tok, OUT_F), jnp.float32),
        compiler_params=pltpu.CompilerParams(
            dimension_semantics=("parallel",)),
    )(my_full, bt)

    return out.reshape(orig_shape[:-1] + (OUT_F,))


# speedup vs baseline: 1.4189x; 1.4189x over previous
import dataclasses
import functools

import jax
import jax.numpy as jnp
from jax import lax
from jax.experimental import pallas as pl
from jax.experimental.pallas import tpu as pltpu
from jax.experimental.pallas import tpu_sc as plsc

IN_F = 4096
OUT_F = 4096
RANK = 64
TOPK = 8
SCALE = 2.0

NCORES = 2
NSUB = 16
NTILES = NCORES * NSUB
LANES = 16

BT = 256
NCHUNK = 4


def _mm1_body(x_ref, at_ref, y_ref):
    y_ref[...] = jnp.dot(x_ref[...].astype(jnp.bfloat16),
                         at_ref[...].astype(jnp.bfloat16),
                         preferred_element_type=jnp.float32)


def _mm2_body(my_ref, bt_ref, o_ref):
    o_ref[...] = jnp.dot(my_ref[...].astype(jnp.bfloat16),
                         bt_ref[...].astype(jnp.bfloat16),
                         preferred_element_type=jnp.float32) * SCALE


def _make_sc_mask(n_tok):
    T = n_tok // NTILES
    G = T // LANES
    mesh = plsc.VectorSubcoreMesh(core_axis_name="c", subcore_axis_name="s")
    cp = pltpu.CompilerParams()
    if "needs_layout_passes" in pltpu.CompilerParams.__dataclass_fields__:
        cp = dataclasses.replace(cp, needs_layout_passes=False)

    @functools.partial(
        pl.kernel,
        mesh=mesh,
        compiler_params=cp,
        out_type=jax.ShapeDtypeStruct((n_tok, RANK), jnp.float32),
        scratch_types=[
            pltpu.VMEM((T, RANK), jnp.float32),
            pltpu.VMEM((RANK, T), jnp.float32),
            pltpu.VMEM((RANK, LANES), jnp.float32),
            pltpu.SemaphoreType.DMA,
        ],
    )
    def sc_mask(y_hbm, d_hbm, o_hbm, y_v, a_t, d_v, sem):
        wid = lax.axis_index("s") * NCORES + lax.axis_index("c")
        base = wid * T
        pltpu.async_copy(y_hbm.at[pl.ds(base, T)], y_v, sem).wait()
        pltpu.async_copy(d_hbm, d_v, sem).wait()
        iota = lax.broadcasted_iota(jnp.int32, (LANES,), 0)

        @pl.loop(0, G)
        def _group(g):
            row0 = g * LANES
            rows = iota + row0

            for r in range(RANK):
                cols = jnp.full((LANES,), r, jnp.int32)
                v = plsc.load_gather(y_v, [rows, cols])
                a_t[r, pl.ds(row0, LANES)] = jnp.abs(v + d_v[r, :])

            for _ in range(TOPK):
                m = jnp.full((LANES,), -1.0, jnp.float32)
                argm = jnp.zeros((LANES,), jnp.int32)
                for b in range(8):
                    mb = a_t[8 * b, pl.ds(row0, LANES)]
                    ab = jnp.full((LANES,), 8 * b, jnp.int32)
                    for j in range(1, 8):
                        v = a_t[8 * b + j, pl.ds(row0, LANES)]
                        gt = v > mb
                        mb = jnp.where(gt, v, mb)
                        ab = jnp.where(gt, 8 * b + j, ab)
                    gt = mb > m
                    m = jnp.where(gt, mb, m)
                    argm = jnp.where(gt, ab, argm)
                plsc.store_scatter(a_t, [argm, rows],
                                   jnp.full((LANES,), -1.0, jnp.float32))

            for r in range(RANK):
                cols = jnp.full((LANES,), r, jnp.int32)
                keep = a_t[r, pl.ds(row0, LANES)] == -1.0
                v = plsc.load_gather(y_v, [rows, cols])
                plsc.store_scatter(y_v, [rows, cols],
                                   jnp.where(keep, v, 0.0))

        pltpu.sync_copy(y_v, o_hbm.at[pl.ds(base, T)])

    return sc_mask


def kernel(x, A, d, B):
    orig_shape = x.shape
    xt = x.reshape(-1, IN_F)
    n_tok = xt.shape[0]
    nt = n_tok // NCHUNK

    at = A.T
    bt = B.T
    d_splat = jnp.broadcast_to(d.reshape(RANK, 1), (RANK, LANES))

    sc_mask = _make_sc_mask(nt)

    masked = []
    for c in range(NCHUNK):
        y_c = pl.pallas_call(
            _mm1_body,
            grid=(nt // BT,),
            in_specs=[
                pl.BlockSpec((BT, IN_F), lambda i, c=c: (c * (nt // BT) + i, 0)),
                pl.BlockSpec((IN_F, RANK), lambda i: (0, 0)),
            ],
            out_specs=pl.BlockSpec((BT, RANK), lambda i: (i, 0)),
            out_shape=jax.ShapeDtypeStruct((nt, RANK), jnp.float32),
            compiler_params=pltpu.CompilerParams(
                dimension_semantics=("parallel",)),
        )(xt, at)
        masked.append(sc_mask(y_c, d_splat))

    my_full = jnp.concatenate(masked, axis=0)

    out = pl.pallas_call(
        _mm2_body,
        grid=(n_tok // BT,),
        in_specs=[
            pl.BlockSpec((BT, RANK), lambda i: (i, 0)),
            pl.BlockSpec((RANK, OUT_F), lambda i: (0, 0)),
        ],
        out_specs=pl.BlockSpec((BT, OUT_F), lambda i: (i, 0)),
        out_shape=jax.ShapeDtypeStruct((n_tok, OUT_F), jnp.float32),
        compiler_params=pltpu.CompilerParams(
            dimension_semantics=("parallel",)),
    )(my_full, bt)

    return out.reshape(orig_shape[:-1] + (OUT_F,))


# speedup vs baseline: 1.4618x; 1.0302x over previous
import dataclasses
import functools

import jax
import jax.numpy as jnp
from jax import lax
from jax.experimental import pallas as pl
from jax.experimental.pallas import tpu as pltpu
from jax.experimental.pallas import tpu_sc as plsc

IN_F = 4096
OUT_F = 4096
RANK = 64
TOPK = 8
SCALE = 2.0

NCORES = 2
NSUB = 16
NTILES = NCORES * NSUB
LANES = 16

BT = 256
NCHUNK = 1


def _mm1_body(x_ref, at_ref, y_ref):
    y_ref[...] = jnp.dot(x_ref[...].astype(jnp.bfloat16),
                         at_ref[...].astype(jnp.bfloat16),
                         preferred_element_type=jnp.float32)


def _mm2_body(my_ref, bt_ref, o_ref):
    o_ref[...] = jnp.dot(my_ref[...].astype(jnp.bfloat16),
                         bt_ref[...].astype(jnp.bfloat16),
                         preferred_element_type=jnp.float32) * SCALE


def _make_sc_mask(n_tok):
    T = n_tok // NTILES
    G = T // LANES
    mesh = plsc.VectorSubcoreMesh(core_axis_name="c", subcore_axis_name="s")
    cp = pltpu.CompilerParams()
    if "needs_layout_passes" in pltpu.CompilerParams.__dataclass_fields__:
        cp = dataclasses.replace(cp, needs_layout_passes=False)

    @functools.partial(
        pl.kernel,
        mesh=mesh,
        compiler_params=cp,
        out_type=jax.ShapeDtypeStruct((n_tok, RANK), jnp.float32),
        scratch_types=[
            pltpu.VMEM((T, RANK), jnp.float32),
            pltpu.VMEM((RANK, T), jnp.float32),
            pltpu.VMEM((RANK, LANES), jnp.float32),
            pltpu.SemaphoreType.DMA,
        ],
    )
    def sc_mask(y_hbm, d_hbm, o_hbm, y_v, a_t, d_v, sem):
        wid = lax.axis_index("s") * NCORES + lax.axis_index("c")
        base = wid * T
        pltpu.async_copy(y_hbm.at[pl.ds(base, T)], y_v, sem).wait()
        pltpu.async_copy(d_hbm, d_v, sem).wait()
        iota = lax.broadcasted_iota(jnp.int32, (LANES,), 0)

        @pl.loop(0, G)
        def _group(g):
            row0 = g * LANES
            rows = iota + row0

            for r in range(RANK):
                cols = jnp.full((LANES,), r, jnp.int32)
                v = plsc.load_gather(y_v, [rows, cols])
                a_t[r, pl.ds(row0, LANES)] = jnp.abs(v + d_v[r, :])

            for _ in range(TOPK):
                m = jnp.full((LANES,), -1.0, jnp.float32)
                argm = jnp.zeros((LANES,), jnp.int32)
                for b in range(8):
                    mb = a_t[8 * b, pl.ds(row0, LANES)]
                    ab = jnp.full((LANES,), 8 * b, jnp.int32)
                    for j in range(1, 8):
                        v = a_t[8 * b + j, pl.ds(row0, LANES)]
                        gt = v > mb
                        mb = jnp.where(gt, v, mb)
                        ab = jnp.where(gt, 8 * b + j, ab)
                    gt = mb > m
                    m = jnp.where(gt, mb, m)
                    argm = jnp.where(gt, ab, argm)
                plsc.store_scatter(a_t, [argm, rows],
                                   jnp.full((LANES,), -1.0, jnp.float32))

            for r in range(RANK):
                cols = jnp.full((LANES,), r, jnp.int32)
                keep = a_t[r, pl.ds(row0, LANES)] == -1.0
                v = plsc.load_gather(y_v, [rows, cols])
                plsc.store_scatter(y_v, [rows, cols],
                                   jnp.where(keep, v, 0.0))

        pltpu.sync_copy(y_v, o_hbm.at[pl.ds(base, T)])

    return sc_mask


def kernel(x, A, d, B):
    orig_shape = x.shape
    xt = x.reshape(-1, IN_F)
    n_tok = xt.shape[0]
    nt = n_tok // NCHUNK

    at = A.T
    bt = B.T
    d_splat = jnp.broadcast_to(d.reshape(RANK, 1), (RANK, LANES))

    sc_mask = _make_sc_mask(nt)

    masked = []
    for c in range(NCHUNK):
        y_c = pl.pallas_call(
            _mm1_body,
            grid=(nt // BT,),
            in_specs=[
                pl.BlockSpec((BT, IN_F), lambda i, c=c: (c * (nt // BT) + i, 0)),
                pl.BlockSpec((IN_F, RANK), lambda i: (0, 0)),
            ],
            out_specs=pl.BlockSpec((BT, RANK), lambda i: (i, 0)),
            out_shape=jax.ShapeDtypeStruct((nt, RANK), jnp.float32),
            compiler_params=pltpu.CompilerParams(
                dimension_semantics=("parallel",)),
        )(xt, at)
        masked.append(sc_mask(y_c, d_splat))

    my_full = jnp.concatenate(masked, axis=0) if NCHUNK > 1 else masked[0]

    out = pl.pallas_call(
        _mm2_body,
        grid=(n_tok // BT,),
        in_specs=[
            pl.BlockSpec((BT, RANK), lambda i: (i, 0)),
            pl.BlockSpec((RANK, OUT_F), lambda i: (0, 0)),
        ],
        out_specs=pl.BlockSpec((BT, OUT_F), lambda i: (i, 0)),
        out_shape=jax.ShapeDtypeStruct((n_tok, OUT_F), jnp.float32),
        compiler_params=pltpu.CompilerParams(
            dimension_semantics=("parallel",)),
    )(my_full, bt)

    return out.reshape(orig_shape[:-1] + (OUT_F,))


# speedup vs baseline: 1.4965x; 1.0237x over previous
import dataclasses
import functools

import jax
import jax.numpy as jnp
from jax import lax
from jax.experimental import pallas as pl
from jax.experimental.pallas import tpu as pltpu
from jax.experimental.pallas import tpu_sc as plsc

IN_F = 4096
OUT_F = 4096
RANK = 64
TOPK = 8
SCALE = 2.0

NCORES = 2
NSUB = 16
NTILES = NCORES * NSUB
LANES = 16

BT = 256
NCHUNK = 1


def _mm1_body(x_ref, at_ref, y_ref):
    y_ref[...] = jnp.dot(x_ref[...].astype(jnp.bfloat16),
                         at_ref[...].astype(jnp.bfloat16),
                         preferred_element_type=jnp.float32)


def _mm2_body(my_ref, bt_ref, o_ref):
    o_ref[...] = jnp.dot(my_ref[...].astype(jnp.bfloat16),
                         bt_ref[...].astype(jnp.bfloat16),
                         preferred_element_type=jnp.float32) * SCALE


def _make_sc_mask(n_tok):
    T = n_tok // NTILES
    G = T // LANES
    mesh = plsc.VectorSubcoreMesh(core_axis_name="c", subcore_axis_name="s")
    cp = pltpu.CompilerParams()
    if "needs_layout_passes" in pltpu.CompilerParams.__dataclass_fields__:
        cp = dataclasses.replace(cp, needs_layout_passes=False)

    @functools.partial(
        pl.kernel,
        mesh=mesh,
        compiler_params=cp,
        out_type=jax.ShapeDtypeStruct((n_tok, RANK), jnp.float32),
        scratch_types=[
            pltpu.VMEM((T, RANK), jnp.float32),
            pltpu.VMEM((RANK * T,), jnp.float32),
            pltpu.VMEM((RANK, LANES), jnp.float32),
            pltpu.SemaphoreType.DMA,
        ],
    )
    def sc_mask(y_hbm, d_hbm, o_hbm, y_v, a_t, d_v, sem):
        wid = lax.axis_index("s") * NCORES + lax.axis_index("c")
        base = wid * T
        pltpu.async_copy(y_hbm.at[pl.ds(base, T)], y_v, sem).wait()
        pltpu.async_copy(d_hbm, d_v, sem).wait()
        iota = lax.broadcasted_iota(jnp.int32, (LANES,), 0)
        neg1 = jnp.full((LANES,), -1.0, jnp.float32)

        @pl.loop(0, G)
        def _group(g):
            row0 = g * LANES
            rows = iota + row0

            mbs, abs_ = [], []
            for b in range(8):
                mb = None
                ab = None
                for j in range(8):
                    r = 8 * b + j
                    cols = jnp.full((LANES,), r, jnp.int32)
                    v = plsc.load_gather(y_v, [rows, cols])
                    a = jnp.abs(v + d_v[r, :])
                    a_t[pl.ds(r * T + row0, LANES)] = a
                    if mb is None:
                        mb, ab = a, jnp.full((LANES,), r, jnp.int32)
                    else:
                        gt = a > mb
                        mb = jnp.where(gt, a, mb)
                        ab = jnp.where(gt, jnp.full((LANES,), r, jnp.int32), ab)
                mbs.append(mb)
                abs_.append(ab)

            for it in range(TOPK):
                m, argm = mbs[0], abs_[0]
                for b in range(1, 8):
                    gt = mbs[b] > m
                    m = jnp.where(gt, mbs[b], m)
                    argm = jnp.where(gt, abs_[b], argm)
                pos = argm * T + rows
                plsc.store_scatter(a_t, [pos], neg1)
                if it == TOPK - 1:
                    break
                blk0 = argm & ~7
                bbase = blk0 * T + rows
                nb = plsc.load_gather(a_t, [bbase])
                nab = blk0
                for j in range(1, 8):
                    v = plsc.load_gather(a_t, [bbase + (j * T)])
                    gt = v > nb
                    nb = jnp.where(gt, v, nb)
                    nab = jnp.where(gt, blk0 + j, nab)
                for b in range(8):
                    upd = blk0 == (8 * b)
                    mbs[b] = jnp.where(upd, nb, mbs[b])
                    abs_[b] = jnp.where(upd, nab, abs_[b])

            for r in range(RANK):
                cols = jnp.full((LANES,), r, jnp.int32)
                keep = a_t[pl.ds(r * T + row0, LANES)] == -1.0
                v = plsc.load_gather(y_v, [rows, cols])
                plsc.store_scatter(y_v, [rows, cols],
                                   jnp.where(keep, v, 0.0))

        pltpu.sync_copy(y_v, o_hbm.at[pl.ds(base, T)])

    return sc_mask


def kernel(x, A, d, B):
    orig_shape = x.shape
    xt = x.reshape(-1, IN_F)
    n_tok = xt.shape[0]
    nt = n_tok // NCHUNK

    at = A.T
    bt = B.T
    d_splat = jnp.broadcast_to(d.reshape(RANK, 1), (RANK, LANES))

    sc_mask = _make_sc_mask(nt)

    masked = []
    for c in range(NCHUNK):
        y_c = pl.pallas_call(
            _mm1_body,
            grid=(nt // BT,),
            in_specs=[
                pl.BlockSpec((BT, IN_F), lambda i, c=c: (c * (nt // BT) + i, 0)),
                pl.BlockSpec((IN_F, RANK), lambda i: (0, 0)),
            ],
            out_specs=pl.BlockSpec((BT, RANK), lambda i: (i, 0)),
            out_shape=jax.ShapeDtypeStruct((nt, RANK), jnp.float32),
            compiler_params=pltpu.CompilerParams(
                dimension_semantics=("parallel",)),
        )(xt, at)
        masked.append(sc_mask(y_c, d_splat))

    my_full = jnp.concatenate(masked, axis=0) if NCHUNK > 1 else masked[0]

    out = pl.pallas_call(
        _mm2_body,
        grid=(n_tok // BT,),
        in_specs=[
            pl.BlockSpec((BT, RANK), lambda i: (i, 0)),
            pl.BlockSpec((RANK, OUT_F), lambda i: (0, 0)),
        ],
        out_specs=pl.BlockSpec((BT, OUT_F), lambda i: (i, 0)),
        out_shape=jax.ShapeDtypeStruct((n_tok, OUT_F), jnp.float32),
        compiler_params=pltpu.CompilerParams(
            dimension_semantics=("parallel",)),
    )(my_full, bt)

    return out.reshape(orig_shape[:-1] + (OUT_F,))


# speedup vs baseline: 2.5550x; 1.7073x over previous
import jax
import jax.numpy as jnp
from jax.experimental import pallas as pl
from jax.experimental.pallas import tpu as pltpu

IN_F = 4096
OUT_F = 4096
RANK = 64
TOPK = 8
SCALE = 2.0


def _fused_body(x_ref, at_ref, d_ref, bt_ref, o_ref):
    xb = x_ref[...]
    y = jnp.dot(xb.astype(jnp.bfloat16), at_ref[...].astype(jnp.bfloat16),
                preferred_element_type=jnp.float32)
    a = jnp.abs(y + d_ref[...])

    bt = a.shape[0]
    work = a.T
    iota = jax.lax.broadcasted_iota(jnp.int32, (RANK, bt), 0)
    keep = jnp.zeros((RANK, bt), jnp.float32)
    for _ in range(TOPK):
        m = jnp.max(work, axis=0, keepdims=True)
        first = jnp.min(jnp.where(work == m, iota, RANK), axis=0, keepdims=True)
        sel = iota == first
        keep = jnp.where(sel, 1.0, keep)
        work = jnp.where(sel, -1.0, work)
    masked_y = y * keep.T

    out = jnp.dot(masked_y.astype(jnp.bfloat16), bt_ref[...].astype(jnp.bfloat16),
                  preferred_element_type=jnp.float32)
    o_ref[...] = out * SCALE


def kernel(x, A, d, B):
    orig_shape = x.shape
    xt = x.reshape(-1, IN_F)
    n_tok = xt.shape[0]
    BT = 256
    grid = (n_tok // BT,)

    out = pl.pallas_call(
        _fused_body,
        grid=grid,
        in_specs=[
            pl.BlockSpec((BT, IN_F), lambda i: (i, 0)),
            pl.BlockSpec((IN_F, RANK), lambda i: (0, 0)),
            pl.BlockSpec((1, RANK), lambda i: (0, 0)),
            pl.BlockSpec((RANK, OUT_F), lambda i: (0, 0)),
        ],
        out_specs=pl.BlockSpec((BT, OUT_F), lambda i: (i, 0)),
        out_shape=jax.ShapeDtypeStruct((n_tok, OUT_F), jnp.float32),
        compiler_params=pltpu.CompilerParams(
            dimension_semantics=("parallel",)),
    )(xt, A.T, d.reshape(1, RANK), B.T)

    return out.reshape(orig_shape[:-1] + (OUT_F,))
